# baseline (speedup 1.0000x reference)
import functools

import jax
import jax.numpy as jnp
from jax import lax
from jax.experimental import pallas as pl
from jax.experimental.pallas import tpu as pltpu
from jax.experimental.pallas import tpu_sc as plsc

_N = 50000
_E = 800000
_NP = 50048
_NT = 16
_NPT = _NP // _NT
_ZR = 46


_G = 125
_G3 = 125
_NBUF = 4
_AHEAD = 3
_SDEPTH = 1
_CB = 20

_mesh = plsc.VectorSubcoreMesh(core_axis_name="c", subcore_axis_name="s")
_sc_params = pltpu.CompilerParams(use_tc_tiling_on_sc=False)


def _zero_acc(zero_v, acc_sh, base, w):
    def zrow(i, carry):
        for j in range(w // 16):
            zero_v[i, pl.ds(16 * j, 16)] = jnp.zeros((16,), jnp.float32)
        return carry

    lax.fori_loop(0, _ZR, zrow, 0)

    def zcopy(i, carry):
        pltpu.sync_copy(zero_v, acc_sh.at[pl.ds(base + i * _ZR, _ZR)])
        return carry

    lax.fori_loop(0, _NPT // _ZR, zcopy, 0)
    plsc.subcore_barrier()


def _make_segsum16():
    w = 16
    ept = _E // 2 // _NT
    ng = ept // _G

    @functools.partial(
        pl.kernel,
        mesh=_mesh,
        compiler_params=_sc_params,
        out_type=jax.ShapeDtypeStruct((2, _NP, w), jnp.float32),
        scratch_types=[
            pltpu.VMEM((ng, _G), jnp.int32),
            pltpu.VMEM((ng, _G), jnp.int32),
            pltpu.VMEM((_NBUF, _G, w), jnp.float32),
            pltpu.VMEM((_ZR, w), jnp.float32),
            pltpu.VMEM_SHARED((_NP, w), jnp.float32),
            pltpu.SemaphoreType.DMA,
            pltpu.SemaphoreType.DMA,
        ],
    )
    def segsum(src_hbm, dst_hbm, val_hbm, out_hbm, src_all, dst_all, rows,
               zero_v, acc_sh, sem_g, sem_s):
        c = lax.axis_index("c")
        tid = lax.axis_index("s")
        base = tid * _NPT
        _zero_acc(zero_v, acc_sh, base, w)

        row0 = c * (_E // 2 // _G) + tid * ng
        pltpu.sync_copy(src_hbm.at[pl.ds(row0, ng)], src_all)
        pltpu.sync_copy(dst_hbm.at[pl.ds(row0, ng)], dst_all)

        def gather(grp, kb):
            pltpu.async_copy(val_hbm.at[src_all.at[grp]], rows.at[kb],
                             sem_g)

        def wait_gather():
            pltpu.make_async_copy(val_hbm.at[src_all.at[0]], rows.at[0],
                                  sem_g).wait()

        def wait_scatter():
            pltpu.make_async_copy(rows.at[0], acc_sh.at[dst_all.at[0]],
                                  sem_s).wait()

        for k in range(_AHEAD):
            gather(k, k)

        def outer(i, carry):
            for k in range(_NBUF):
                grp = i * _NBUF + k
                wait_gather()
                pltpu.async_copy(rows.at[k], acc_sh.at[dst_all.at[grp]],
                                 sem_s, add=True)

                @pl.when(grp >= _SDEPTH)
                def _():
                    wait_scatter()

                @pl.when(grp + _AHEAD < ng)
                def _():
                    gather(grp + _AHEAD, (k + _AHEAD) % _NBUF)
            return carry

        lax.fori_loop(0, ng // _NBUF, outer, 0)
        for _ in range(_SDEPTH):
            wait_scatter()
        plsc.subcore_barrier()
        pltpu.sync_copy(acc_sh.at[pl.ds(base, _NPT)],
                        out_hbm.at[c, pl.ds(base, _NPT)])

    return segsum


def _make_segsum32():
    w = 32
    ept = _E // _NT
    ng = ept // _G3
    nc = ng // _CB
    assert _CB % _NBUF == 0

    @functools.partial(
        pl.kernel,
        mesh=_mesh,
        compiler_params=_sc_params,
        out_type=[
            jax.ShapeDtypeStruct((_NP, w), jnp.float32),
            jax.ShapeDtypeStruct((_NP, w), jnp.float32),
        ],
        scratch_types=[
            pltpu.VMEM((2, _CB, _G3), jnp.int32),
            pltpu.VMEM((2, _CB, _G3), jnp.int32),
            pltpu.VMEM((_NBUF, _G3, w), jnp.float32),
            pltpu.VMEM((_ZR, w), jnp.float32),
            pltpu.VMEM_SHARED((_NP, w), jnp.float32),
            pltpu.SemaphoreType.DMA,
            pltpu.SemaphoreType.DMA,
            pltpu.SemaphoreType.DMA,
        ],
    )
    def segsum(src_hbm, dst_hbm, val0_hbm, val1_hbm, out0_hbm, out1_hbm,
               srcb, dstb, rows, zero_v, acc_sh, sem_g, sem_s, sem_i):
        c = lax.axis_index("c")
        tid = lax.axis_index("s")
        base = tid * _NPT
        _zero_acc(zero_v, acc_sh, base, w)

        row0 = tid * ng

        def load_idx(chunk, buf):
            pltpu.async_copy(src_hbm.at[pl.ds(row0 + chunk * _CB, _CB)],
                             srcb.at[buf], sem_i)
            pltpu.async_copy(dst_hbm.at[pl.ds(row0 + chunk * _CB, _CB)],
                             dstb.at[buf], sem_i)

        def wait_idx():
            pltpu.make_async_copy(src_hbm.at[pl.ds(0, _CB)], srcb.at[0],
                                  sem_i).wait()
            pltpu.make_async_copy(dst_hbm.at[pl.ds(0, _CB)], dstb.at[0],
                                  sem_i).wait()

        def run(val_hbm, out_hbm):
            def gather(buf, slot, kb):
                pltpu.async_copy(val_hbm.at[srcb.at[buf, slot]],
                                 rows.at[kb], sem_g)

            def wait_gather():
                pltpu.make_async_copy(val_hbm.at[srcb.at[0, 0]],
                                      rows.at[0], sem_g).wait()

            def wait_scatter():
                pltpu.make_async_copy(rows.at[0], acc_sh.at[dstb.at[0, 0]],
                                      sem_s).wait()

            def slot(grp, cur, nxt, s, load_fn, wait_fn, tail_gather):
                if s == 2 and load_fn is not None:
                    load_fn()
                if s == _CB - _AHEAD and wait_fn is not None:
                    wait_fn()
                wait_gather()
                pltpu.async_copy(rows.at[s % _NBUF],
                                 acc_sh.at[dstb.at[cur, s]], sem_s, add=True)

                @pl.when(grp >= _SDEPTH)
                def _():
                    wait_scatter()

                kb = (s + _AHEAD) % _NBUF
                if s < _CB - _AHEAD:
                    gather(cur, s + _AHEAD, kb)
                else:
                    tail_gather(nxt, s + _AHEAD - _CB, kb)

            pltpu.sync_copy(src_hbm.at[pl.ds(row0, _CB)], srcb.at[0])
            pltpu.sync_copy(dst_hbm.at[pl.ds(row0, _CB)], dstb.at[0])
            load_idx(1, 1)
            for k in range(_AHEAD):
                gather(0, k, k)

            par = nc % 2
            npairs = (nc - par) // 2
            if par:
                for s in range(_CB):
                    slot(s, 0, 1, s, None,
                         wait_idx if s == _CB - _AHEAD else None, gather)

            def outer(i, carry):
                for half in range(2):
                    m_off = par + half
                    cur = m_off % 2
                    nxt = 1 - cur
                    for s in range(_CB):
                        grp = (2 * i + m_off) * _CB + s
                        if m_off == 0:
                            def load_fn(i=i):
                                @pl.when(i > 0)
                                def _():
                                    load_idx(2 * i + 1, 1)

                            wait_fn = wait_idx
                            tail = gather
                        elif half == 0:
                            def load_fn(i=i, m_off=m_off):
                                load_idx(2 * i + m_off + 1, 1 - m_off % 2)

                            wait_fn = wait_idx
                            tail = gather
                        else:
                            def load_fn(i=i, m_off=m_off):
                                @pl.when(i < npairs - 1)
                                def _():
                                    load_idx(2 * i + m_off + 1, 1 - m_off % 2)

                            def wait_fn(i=i):
                                @pl.when(i < npairs - 1)
                                def _():
                                    wait_idx()

                            def tail(b, sl, kb, i=i):
                                @pl.when(i < npairs - 1)
                                def _():
                                    gather(b, sl, kb)
                        slot(grp, cur, nxt, s, load_fn, wait_fn, tail)
                return carry

            lax.fori_loop(0, npairs, outer, 0)
            for _ in range(_SDEPTH):
                wait_scatter()
            plsc.subcore_barrier()
            pltpu.sync_copy(acc_sh.at[pl.ds(base, _NPT)],
                            out_hbm.at[pl.ds(base, _NPT)])

        @pl.when(c == 0)
        def _():
            run(val0_hbm, out0_hbm)

        @pl.when(c == 1)
        def _():
            run(val1_hbm, out1_hbm)

    return segsum


_R = 2000
_GRID = _N // _R


def _rows_spec(width):
    return pl.BlockSpec((_R, width), lambda i: (i, 0))


def _full_spec(shape):
    return pl.BlockSpec(shape, lambda i: (0, 0))


def _dot(a, b):
    return jnp.dot(a, b, preferred_element_type=jnp.float32)


def _t0(x, wl0x, wr0x):
    def body(xr, wl, wr, xlo, xro):
        xlo[...] = _dot(xr[...], wl[...])
        xro[...] = _dot(xr[...], wr[...])

    return pl.pallas_call(
        body,
        grid=(_GRID,),
        in_specs=[_rows_spec(128), _full_spec((128, 64)),
                  _full_spec((128, 64))],
        out_specs=[_rows_spec(64), _rows_spec(64)],
        out_shape=[
            jax.ShapeDtypeStruct((_N, 64), jnp.float32),
            jax.ShapeDtypeStruct((_N, 64), jnp.float32),
        ],
    )(x, wl0x, wr0x)


def _flat_spec(width):
    return pl.BlockSpec((_R * width,), lambda i: (i,))


def _t1(parts, x1, gin_w0, gin_b0, gin_w1):
    def body(pp, x1r, w0, b0, w1, u0, u1, degr):
        a1 = pp[0, :, 0:1] + pp[1, :, 0:1]
        deg = jnp.clip(pp[0, :, 1:2] + pp[1, :, 1:2], 1.0, None)
        g1 = jnp.tanh((x1r[...] + a1) * w0[0:1, :] + b0[...])
        u = _dot(g1, w1[...])
        u0[...] = u[:, :32]
        u1[...] = u[:, 32:]
        degr[...] = deg

    return pl.pallas_call(
        body,
        grid=(_GRID,),
        in_specs=[
            pl.BlockSpec((2, _R, 16), lambda i: (0, i, 0)), _rows_spec(1),
            _full_spec((1, 64)), _full_spec((1, 64)), _full_spec((64, 64)),
        ],
        out_specs=[_rows_spec(32), _rows_spec(32), _rows_spec(1)],
        out_shape=[
            jax.ShapeDtypeStruct((_N, 32), jnp.float32),
            jax.ShapeDtypeStruct((_N, 32), jnp.float32),
            jax.ShapeDtypeStruct((_N, 1), jnp.float32),
        ],
    )(parts, x1, gin_w0, gin_b0, gin_w1)


def _t2(xl, xr, u0, u1, a20, a21, gin_b1, wl0g, wr0g, bl0):
    def body(xlr, xrr, u0r, u1r, a0r, a1r, b1, wlg, wrg, b, p0, p1, rr):
        u = jnp.concatenate([u0r[...], u1r[...]], axis=1)
        a2 = jnp.concatenate([a0r[...], a1r[...]], axis=1)
        g2 = jnp.tanh(u + a2 + b1[...])
        p = xlr[...] + _dot(g2, wlg[...])
        r = xrr[...] + _dot(g2, wrg[...]) + b[...]
        p0[...] = p[:, :32]
        p1[...] = p[:, 32:]
        rr[...] = r

    return pl.pallas_call(
        body,
        grid=(_GRID,),
        in_specs=[
            _rows_spec(64), _rows_spec(64), _rows_spec(32), _rows_spec(32),
            _rows_spec(32), _rows_spec(32),
            _full_spec((1, 64)), _full_spec((64, 64)), _full_spec((64, 64)),
            _full_spec((1, 64)),
        ],
        out_specs=[_rows_spec(32), _rows_spec(32), _rows_spec(64)],
        out_shape=[
            jax.ShapeDtypeStruct((_N, 32), jnp.float32),
            jax.ShapeDtypeStruct((_N, 32), jnp.float32),
            jax.ShapeDtypeStruct((_N, 64), jnp.float32),
        ],
    )(xl, xr, u0, u1, a20, a21, gin_b1, wl0g, wr0g, bl0)


def _t3(sp0, sp1, deg, rp, wl1, wr1, bl1):
    def body(s0, s1, degr, rpr, wl, wr, b, q0, q1, r1r):
        sp = jnp.concatenate([s0[...], s1[...]], axis=1)
        s = jax.nn.relu(sp / degr[...] + rpr[...])
        q = _dot(s, wl[...])
        r1 = _dot(s, wr[...]) + b[...]
        q0[...] = q[:, :32]
        q1[...] = q[:, 32:]
        r1r[...] = r1

    return pl.pallas_call(
        body,
        grid=(_GRID,),
        in_specs=[
            _rows_spec(32), _rows_spec(32), _rows_spec(1), _rows_spec(64),
            _full_spec((64, 64)), _full_spec((64, 64)), _full_spec((1, 64)),
        ],
        out_specs=[_rows_spec(32), _rows_spec(32), _rows_spec(64)],
        out_shape=[
            jax.ShapeDtypeStruct((_N, 32), jnp.float32),
            jax.ShapeDtypeStruct((_N, 32), jnp.float32),
            jax.ShapeDtypeStruct((_N, 64), jnp.float32),
        ],
    )(sp0, sp1, deg, rp, wl1, wr1, bl1)


def _t4(sq0, sq1, deg, r1p, out_w, out_b):
    def body(s0, s1, degr, rr, w, b, o):
        sq = jnp.concatenate([s0[...], s1[...]], axis=1)
        t = jax.nn.relu(sq / degr[...] + rr[...])
        o[...] = _dot(t, w[...]) + b[...]

    return pl.pallas_call(
        body,
        grid=(_GRID,),
        in_specs=[
            _rows_spec(32), _rows_spec(32), _rows_spec(1), _rows_spec(64),
            _full_spec((64, 64)), _full_spec((1, 64)),
        ],
        out_specs=_rows_spec(64),
        out_shape=jax.ShapeDtypeStruct((_N, 64), jnp.float32),
    )(sq0, sq1, deg, r1p, out_w, out_b)


def kernel(x, x1, edge_index, gin_w0, gin_b0, gin_w1, gin_b1,
           sage_wl0, sage_bl0, sage_wr0, sage_wl1, sage_bl1, sage_wr1,
           out_w, out_b):
    src2 = edge_index[0].reshape(_E // _G, _G)
    dst2 = edge_index[1].reshape(_E // _G, _G)

    xl, xr = _t0(x, sage_wl0[:128], sage_wr0[:128])

    vals16 = jnp.concatenate(
        [x1, jnp.ones((_N, 1), jnp.float32), jnp.zeros((_N, 14), jnp.float32)],
        axis=1)
    parts = _make_segsum16()(src2, dst2, vals16)

    u0, u1, deg = _t1(parts, x1, gin_w0, gin_b0.reshape(1, 64), gin_w1)

    seg32 = _make_segsum32()
    a20, a21 = seg32(src2, dst2, u0, u1)

    p0, p1, rp = _t2(xl, xr, u0, u1, a20, a21, gin_b1.reshape(1, 64),
                     sage_wl0[128:], sage_wr0[128:], sage_bl0.reshape(1, 64))

    sp0, sp1 = seg32(src2, dst2, p0, p1)

    q0, q1, r1p = _t3(sp0, sp1, deg, rp, sage_wl1, sage_wr1,
                      sage_bl1.reshape(1, 64))

    sq0, sq1 = seg32(src2, dst2, q0, q1)

    return _t4(sq0, sq1, deg, r1p, out_w, out_b.reshape(1, 64))


# speedup vs baseline: 1.0089x; 1.0089x over previous
import functools

import jax
import jax.numpy as jnp
from jax import lax
from jax.experimental import pallas as pl
from jax.experimental.pallas import tpu as pltpu
from jax.experimental.pallas import tpu_sc as plsc

_N = 50000
_E = 800000
_NP = 50048
_NT = 16
_NPT = _NP // _NT
_ZR = 136


_G = 100
_G3 = 100
_NBUF = 5
_AHEAD = 4
_SDEPTH = 1
_CB = 25

_mesh = plsc.VectorSubcoreMesh(core_axis_name="c", subcore_axis_name="s")
_sc_params = pltpu.CompilerParams(use_tc_tiling_on_sc=False)


def _zero_acc(zero_v, acc_sh, base, w):
    def zrow(i, carry):
        for j in range(w // 16):
            zero_v[i, pl.ds(16 * j, 16)] = jnp.zeros((16,), jnp.float32)
        return carry

    lax.fori_loop(0, _ZR, zrow, 0)

    def zcopy(i, carry):
        pltpu.sync_copy(zero_v, acc_sh.at[pl.ds(base + i * _ZR, _ZR)])
        return carry

    lax.fori_loop(0, _NPT // _ZR, zcopy, 0)
    plsc.subcore_barrier()


def _make_segsum16():
    w = 16
    ept = _E // 2 // _NT
    ng = ept // _G

    @functools.partial(
        pl.kernel,
        mesh=_mesh,
        compiler_params=_sc_params,
        out_type=jax.ShapeDtypeStruct((2, _NP, w), jnp.float32),
        scratch_types=[
            pltpu.VMEM((ng, _G), jnp.int32),
            pltpu.VMEM((ng, _G), jnp.int32),
            pltpu.VMEM((_NBUF, _G, w), jnp.float32),
            pltpu.VMEM((_ZR, w), jnp.float32),
            pltpu.VMEM_SHARED((_NP, w), jnp.float32),
            pltpu.SemaphoreType.DMA,
            pltpu.SemaphoreType.DMA,
        ],
    )
    def segsum(src_hbm, dst_hbm, val_hbm, out_hbm, src_all, dst_all, rows,
               zero_v, acc_sh, sem_g, sem_s):
        c = lax.axis_index("c")
        tid = lax.axis_index("s")
        base = tid * _NPT
        _zero_acc(zero_v, acc_sh, base, w)

        row0 = c * (_E // 2 // _G) + tid * ng
        pltpu.sync_copy(src_hbm.at[pl.ds(row0, ng)], src_all)
        pltpu.sync_copy(dst_hbm.at[pl.ds(row0, ng)], dst_all)

        def gather(grp, kb):
            pltpu.async_copy(val_hbm.at[src_all.at[grp]], rows.at[kb],
                             sem_g)

        def wait_gather():
            pltpu.make_async_copy(val_hbm.at[src_all.at[0]], rows.at[0],
                                  sem_g).wait()

        def wait_scatter():
            pltpu.make_async_copy(rows.at[0], acc_sh.at[dst_all.at[0]],
                                  sem_s).wait()

        for k in range(_AHEAD):
            gather(k, k)

        def outer(i, carry):
            for k in range(_NBUF):
                grp = i * _NBUF + k
                wait_gather()
                pltpu.async_copy(rows.at[k], acc_sh.at[dst_all.at[grp]],
                                 sem_s, add=True)

                @pl.when(grp >= _SDEPTH)
                def _():
                    wait_scatter()

                @pl.when(grp + _AHEAD < ng)
                def _():
                    gather(grp + _AHEAD, (k + _AHEAD) % _NBUF)
            return carry

        lax.fori_loop(0, ng // _NBUF, outer, 0)
        for _ in range(_SDEPTH):
            wait_scatter()
        plsc.subcore_barrier()
        pltpu.sync_copy(acc_sh.at[pl.ds(base, _NPT)],
                        out_hbm.at[c, pl.ds(base, _NPT)])

    return segsum


def _make_segsum32():
    w = 32
    ept = _E // _NT
    ng = ept // _G3
    nc = ng // _CB
    assert _CB % _NBUF == 0

    @functools.partial(
        pl.kernel,
        mesh=_mesh,
        compiler_params=_sc_params,
        out_type=[
            jax.ShapeDtypeStruct((_NP, w), jnp.float32),
            jax.ShapeDtypeStruct((_NP, w), jnp.float32),
        ],
        scratch_types=[
            pltpu.VMEM((2, _CB, _G3), jnp.int32),
            pltpu.VMEM((2, _CB, _G3), jnp.int32),
            pltpu.VMEM((_NBUF, _G3, w), jnp.float32),
            pltpu.VMEM((_ZR, w), jnp.float32),
            pltpu.VMEM_SHARED((_NP, w), jnp.float32),
            pltpu.SemaphoreType.DMA,
            pltpu.SemaphoreType.DMA,
            pltpu.SemaphoreType.DMA,
        ],
    )
    def segsum(src_hbm, dst_hbm, val0_hbm, val1_hbm, out0_hbm, out1_hbm,
               srcb, dstb, rows, zero_v, acc_sh, sem_g, sem_s, sem_i):
        c = lax.axis_index("c")
        tid = lax.axis_index("s")
        base = tid * _NPT
        _zero_acc(zero_v, acc_sh, base, w)

        row0 = tid * ng

        def load_idx(chunk, buf):
            pltpu.async_copy(src_hbm.at[pl.ds(row0 + chunk * _CB, _CB)],
                             srcb.at[buf], sem_i)
            pltpu.async_copy(dst_hbm.at[pl.ds(row0 + chunk * _CB, _CB)],
                             dstb.at[buf], sem_i)

        def wait_idx():
            pltpu.make_async_copy(src_hbm.at[pl.ds(0, _CB)], srcb.at[0],
                                  sem_i).wait()
            pltpu.make_async_copy(dst_hbm.at[pl.ds(0, _CB)], dstb.at[0],
                                  sem_i).wait()

        def run(val_hbm, out_hbm):
            def gather(buf, slot, kb):
                pltpu.async_copy(val_hbm.at[srcb.at[buf, slot]],
                                 rows.at[kb], sem_g)

            def wait_gather():
                pltpu.make_async_copy(val_hbm.at[srcb.at[0, 0]],
                                      rows.at[0], sem_g).wait()

            def wait_scatter():
                pltpu.make_async_copy(rows.at[0], acc_sh.at[dstb.at[0, 0]],
                                      sem_s).wait()

            def slot(grp, cur, nxt, s, load_fn, wait_fn, tail_gather):
                if s == 2 and load_fn is not None:
                    load_fn()
                if s == _CB - _AHEAD and wait_fn is not None:
                    wait_fn()
                wait_gather()
                pltpu.async_copy(rows.at[s % _NBUF],
                                 acc_sh.at[dstb.at[cur, s]], sem_s, add=True)

                @pl.when(grp >= _SDEPTH)
                def _():
                    wait_scatter()

                kb = (s + _AHEAD) % _NBUF
                if s < _CB - _AHEAD:
                    gather(cur, s + _AHEAD, kb)
                else:
                    tail_gather(nxt, s + _AHEAD - _CB, kb)

            pltpu.sync_copy(src_hbm.at[pl.ds(row0, _CB)], srcb.at[0])
            pltpu.sync_copy(dst_hbm.at[pl.ds(row0, _CB)], dstb.at[0])
            load_idx(1, 1)
            for k in range(_AHEAD):
                gather(0, k, k)

            par = nc % 2
            npairs = (nc - par) // 2
            if par:
                for s in range(_CB):
                    slot(s, 0, 1, s, None,
                         wait_idx if s == _CB - _AHEAD else None, gather)

            def outer(i, carry):
                for half in range(2):
                    m_off = par + half
                    cur = m_off % 2
                    nxt = 1 - cur
                    for s in range(_CB):
                        grp = (2 * i + m_off) * _CB + s
                        if m_off == 0:
                            def load_fn(i=i):
                                @pl.when(i > 0)
                                def _():
                                    load_idx(2 * i + 1, 1)

                            wait_fn = wait_idx
                            tail = gather
                        elif half == 0:
                            def load_fn(i=i, m_off=m_off):
                                load_idx(2 * i + m_off + 1, 1 - m_off % 2)

                            wait_fn = wait_idx
                            tail = gather
                        else:
                            def load_fn(i=i, m_off=m_off):
                                @pl.when(i < npairs - 1)
                                def _():
                                    load_idx(2 * i + m_off + 1, 1 - m_off % 2)

                            def wait_fn(i=i):
                                @pl.when(i < npairs - 1)
                                def _():
                                    wait_idx()

                            def tail(b, sl, kb, i=i):
                                @pl.when(i < npairs - 1)
                                def _():
                                    gather(b, sl, kb)
                        slot(grp, cur, nxt, s, load_fn, wait_fn, tail)
                return carry

            lax.fori_loop(0, npairs, outer, 0)
            for _ in range(_SDEPTH):
                wait_scatter()
            plsc.subcore_barrier()
            pltpu.sync_copy(acc_sh.at[pl.ds(base, _NPT)],
                            out_hbm.at[pl.ds(base, _NPT)])

        @pl.when(c == 0)
        def _():
            run(val0_hbm, out0_hbm)

        @pl.when(c == 1)
        def _():
            run(val1_hbm, out1_hbm)

    return segsum


_R = 2000
_GRID = _N // _R


def _rows_spec(width):
    return pl.BlockSpec((_R, width), lambda i: (i, 0))


def _full_spec(shape):
    return pl.BlockSpec(shape, lambda i: (0, 0))


def _dot(a, b):
    return jnp.dot(a, b, preferred_element_type=jnp.float32)


def _t0(x, wl0x, wr0x):
    def body(xr, wl, wr, xlo, xro):
        xlo[...] = _dot(xr[...], wl[...])
        xro[...] = _dot(xr[...], wr[...])

    return pl.pallas_call(
        body,
        grid=(_GRID,),
        in_specs=[_rows_spec(128), _full_spec((128, 64)),
                  _full_spec((128, 64))],
        out_specs=[_rows_spec(64), _rows_spec(64)],
        out_shape=[
            jax.ShapeDtypeStruct((_N, 64), jnp.float32),
            jax.ShapeDtypeStruct((_N, 64), jnp.float32),
        ],
    )(x, wl0x, wr0x)


def _flat_spec(width):
    return pl.BlockSpec((_R * width,), lambda i: (i,))


def _t1(parts, x1, gin_w0, gin_b0, gin_w1):
    def body(pp, x1r, w0, b0, w1, u0, u1, degr):
        a1 = pp[0, :, 0:1] + pp[1, :, 0:1]
        deg = jnp.clip(pp[0, :, 1:2] + pp[1, :, 1:2], 1.0, None)
        g1 = jnp.tanh((x1r[...] + a1) * w0[0:1, :] + b0[...])
        u = _dot(g1, w1[...])
        u0[...] = u[:, :32]
        u1[...] = u[:, 32:]
        degr[...] = deg

    return pl.pallas_call(
        body,
        grid=(_GRID,),
        in_specs=[
            pl.BlockSpec((2, _R, 16), lambda i: (0, i, 0)), _rows_spec(1),
            _full_spec((1, 64)), _full_spec((1, 64)), _full_spec((64, 64)),
        ],
        out_specs=[_rows_spec(32), _rows_spec(32), _rows_spec(1)],
        out_shape=[
            jax.ShapeDtypeStruct((_N, 32), jnp.float32),
            jax.ShapeDtypeStruct((_N, 32), jnp.float32),
            jax.ShapeDtypeStruct((_N, 1), jnp.float32),
        ],
    )(parts, x1, gin_w0, gin_b0, gin_w1)


def _t2(xl, xr, u0, u1, a20, a21, gin_b1, wl0g, wr0g, bl0):
    def body(xlr, xrr, u0r, u1r, a0r, a1r, b1, wlg, wrg, b, p0, p1, rr):
        u = jnp.concatenate([u0r[...], u1r[...]], axis=1)
        a2 = jnp.concatenate([a0r[...], a1r[...]], axis=1)
        g2 = jnp.tanh(u + a2 + b1[...])
        p = xlr[...] + _dot(g2, wlg[...])
        r = xrr[...] + _dot(g2, wrg[...]) + b[...]
        p0[...] = p[:, :32]
        p1[...] = p[:, 32:]
        rr[...] = r

    return pl.pallas_call(
        body,
        grid=(_GRID,),
        in_specs=[
            _rows_spec(64), _rows_spec(64), _rows_spec(32), _rows_spec(32),
            _rows_spec(32), _rows_spec(32),
            _full_spec((1, 64)), _full_spec((64, 64)), _full_spec((64, 64)),
            _full_spec((1, 64)),
        ],
        out_specs=[_rows_spec(32), _rows_spec(32), _rows_spec(64)],
        out_shape=[
            jax.ShapeDtypeStruct((_N, 32), jnp.float32),
            jax.ShapeDtypeStruct((_N, 32), jnp.float32),
            jax.ShapeDtypeStruct((_N, 64), jnp.float32),
        ],
    )(xl, xr, u0, u1, a20, a21, gin_b1, wl0g, wr0g, bl0)


def _t3(sp0, sp1, deg, rp, wl1, wr1, bl1):
    def body(s0, s1, degr, rpr, wl, wr, b, q0, q1, r1r):
        sp = jnp.concatenate([s0[...], s1[...]], axis=1)
        s = jax.nn.relu(sp / degr[...] + rpr[...])
        q = _dot(s, wl[...])
        r1 = _dot(s, wr[...]) + b[...]
        q0[...] = q[:, :32]
        q1[...] = q[:, 32:]
        r1r[...] = r1

    return pl.pallas_call(
        body,
        grid=(_GRID,),
        in_specs=[
            _rows_spec(32), _rows_spec(32), _rows_spec(1), _rows_spec(64),
            _full_spec((64, 64)), _full_spec((64, 64)), _full_spec((1, 64)),
        ],
        out_specs=[_rows_spec(32), _rows_spec(32), _rows_spec(64)],
        out_shape=[
            jax.ShapeDtypeStruct((_N, 32), jnp.float32),
            jax.ShapeDtypeStruct((_N, 32), jnp.float32),
            jax.ShapeDtypeStruct((_N, 64), jnp.float32),
        ],
    )(sp0, sp1, deg, rp, wl1, wr1, bl1)


def _t4(sq0, sq1, deg, r1p, out_w, out_b):
    def body(s0, s1, degr, rr, w, b, o):
        sq = jnp.concatenate([s0[...], s1[...]], axis=1)
        t = jax.nn.relu(sq / degr[...] + rr[...])
        o[...] = _dot(t, w[...]) + b[...]

    return pl.pallas_call(
        body,
        grid=(_GRID,),
        in_specs=[
            _rows_spec(32), _rows_spec(32), _rows_spec(1), _rows_spec(64),
            _full_spec((64, 64)), _full_spec((1, 64)),
        ],
        out_specs=_rows_spec(64),
        out_shape=jax.ShapeDtypeStruct((_N, 64), jnp.float32),
    )(sq0, sq1, deg, r1p, out_w, out_b)


def kernel(x, x1, edge_index, gin_w0, gin_b0, gin_w1, gin_b1,
           sage_wl0, sage_bl0, sage_wr0, sage_wl1, sage_bl1, sage_wr1,
           out_w, out_b):
    src2 = edge_index[0].reshape(_E // _G, _G)
    dst2 = edge_index[1].reshape(_E // _G, _G)

    xl, xr = _t0(x, sage_wl0[:128], sage_wr0[:128])

    vals16 = jnp.concatenate(
        [x1, jnp.ones((_N, 1), jnp.float32), jnp.zeros((_N, 14), jnp.float32)],
        axis=1)
    parts = _make_segsum16()(src2, dst2, vals16)

    u0, u1, deg = _t1(parts, x1, gin_w0, gin_b0.reshape(1, 64), gin_w1)

    seg32 = _make_segsum32()
    a20, a21 = seg32(src2, dst2, u0, u1)

    p0, p1, rp = _t2(xl, xr, u0, u1, a20, a21, gin_b1.reshape(1, 64),
                     sage_wl0[128:], sage_wr0[128:], sage_bl0.reshape(1, 64))

    sp0, sp1 = seg32(src2, dst2, p0, p1)

    q0, q1, r1p = _t3(sp0, sp1, deg, rp, sage_wl1, sage_wr1,
                      sage_bl1.reshape(1, 64))

    sq0, sq1 = seg32(src2, dst2, q0, q1)

    return _t4(sq0, sq1, deg, r1p, out_w, out_b.reshape(1, 64))
